# baseline (speedup 1.0000x reference)
import functools

import jax
import jax.numpy as jnp
from jax import lax
from jax.experimental import pallas as pl
from jax.experimental.pallas import tpu as pltpu
from jax.experimental.pallas import tpu_sc as plsc

_B = 16384
_K = 8192
_D = 64
_TB = 256
_BETA = 0.25

_NC = 2
_NS = 16
_NW = _NC * _NS
_RW = _B // _NW
_CH = 128


def _normalize_rows(x, eps=1e-12):
    norm = jnp.linalg.norm(x, axis=1, keepdims=True)
    return x / jnp.maximum(norm, eps)


def _tc_body(nh_ref, nw_ref,
             onehot_ref, idx_ref, perp_ref,
             dscr_ref, counts_ref):
    b = pl.program_id(0)
    nb = pl.num_programs(0)

    @pl.when(b == 0)
    def _init():
        counts_ref[...] = jnp.zeros_like(counts_ref)
        perp_ref[...] = jnp.zeros_like(perp_ref)

    dscr_ref[pl.ds((b % 2) * _TB, _TB), :] = jax.lax.dot_general(
        nh_ref[...], nw_ref[...], (((1,), (1,)), ((), ())),
        preferred_element_type=jnp.float32)

    d = dscr_ref[pl.ds(((b + 1) % 2) * _TB, _TB), :]
    m = jnp.max(d, axis=1, keepdims=True)
    col = jax.lax.broadcasted_iota(jnp.int32, (_TB, _K), 1)
    sel = jnp.where(d >= m, col, -1)
    idx = jnp.max(sel, axis=1)
    idx_ref[...] = idx
    oh = (col == idx[:, None]).astype(jnp.float32)
    onehot_ref[...] = oh

    csum = jnp.sum(oh, axis=0, keepdims=True)
    counts_ref[...] += jnp.where(b > 0, csum, jnp.zeros_like(csum))

    @pl.when(b == nb - 1)
    def _fin():
        p = counts_ref[...] * (1.0 / _B)
        ent = jnp.sum(p * jnp.log(p + 1e-10))
        perp_ref[...] = jnp.exp(-ent).reshape(1, 1)


def _tc_call(nh_bf, nw_bf):
    nblk = _B // _TB
    return pl.pallas_call(
        _tc_body,
        grid=(nblk + 1,),
        in_specs=[
            pl.BlockSpec((_TB, _D), lambda b: (jnp.minimum(b, nblk - 1), 0)),
            pl.BlockSpec((_K, _D), lambda b: (0, 0)),
        ],
        out_specs=[
            pl.BlockSpec((_TB, _K), lambda b: (jnp.maximum(b - 1, 0), 0)),
            pl.BlockSpec((_TB,), lambda b: (jnp.maximum(b - 1, 0),)),
            pl.BlockSpec((1, 1), lambda b: (0, 0)),
        ],
        out_shape=[
            jax.ShapeDtypeStruct((_B, _K), jnp.float32),
            jax.ShapeDtypeStruct((_B,), jnp.int32),
            jax.ShapeDtypeStruct((1, 1), jnp.float32),
        ],
        scratch_shapes=[
            pltpu.VMEM((2 * _TB, _K), jnp.float32),
            pltpu.VMEM((1, _K), jnp.float32),
        ],
        compiler_params=pltpu.CompilerParams(
            dimension_semantics=("arbitrary",),
        ),
    )(nh_bf, nw_bf)


def _sc_body(wb_hbm, idx_hbm, h_hbm,
             zqout_hbm, part_hbm,
             idx_v, rows_v, h_v, out_v, part_v, sem):
    wid = lax.axis_index("s") * _NC + lax.axis_index("c")
    base = wid * _RW
    pltpu.sync_copy(idx_hbm.at[pl.ds(base, _RW)], idx_v)

    acc0 = jnp.zeros((16,), jnp.float32)

    def chunk(c, acc):
        cb = base + c * _CH
        pltpu.async_copy(
            wb_hbm.at[idx_v.at[pl.ds(c * _CH, _CH)]], rows_v, sem).wait()
        pltpu.sync_copy(h_hbm.at[pl.ds(cb, _CH)], h_v)

        def row(i, a):
            for j in range(_D // 16):
                zq = rows_v[i, pl.ds(j * 16, 16)]
                hh = h_v[i, pl.ds(j * 16, 16)]
                dd = zq - hh
                out_v[i, pl.ds(j * 16, 16)] = hh + dd
                a = a + dd * dd
            return a

        acc = lax.fori_loop(0, _CH, row, acc)
        pltpu.sync_copy(out_v, zqout_hbm.at[pl.ds(cb, _CH)])
        return acc

    acc = lax.fori_loop(0, _RW // _CH, chunk, acc0)
    part_v[...] = acc
    pltpu.sync_copy(part_v, part_hbm.at[wid])


def _sc_call(wb, idx, h_batch):
    sc = functools.partial(
        pl.kernel,
        out_type=[
            jax.ShapeDtypeStruct((_B, _D), jnp.float32),
            jax.ShapeDtypeStruct((_NW, 16), jnp.float32),
        ],
        mesh=plsc.VectorSubcoreMesh(core_axis_name="c",
                                    subcore_axis_name="s"),
        scratch_types=[
            pltpu.VMEM((_RW,), jnp.int32),
            pltpu.VMEM((_CH, 128), jnp.float32),
            pltpu.VMEM((_CH, _D), jnp.float32),
            pltpu.VMEM((_CH, _D), jnp.float32),
            pltpu.VMEM((16,), jnp.float32),
            pltpu.SemaphoreType.DMA,
        ],
    )(_sc_body)
    return sc(wb, idx, h_batch)


def kernel(h_batch, W):
    nh = _normalize_rows(jax.lax.stop_gradient(h_batch))
    nw = _normalize_rows(W)
    nh_bf = nh.astype(jnp.bfloat16)
    nw_bf = nw.astype(jnp.bfloat16)
    wb = W.astype(jnp.bfloat16).astype(jnp.float32)
    wb = jnp.concatenate([wb, jnp.zeros((_K, 128 - _D), jnp.float32)], axis=1)

    onehot, idx, perp = _tc_call(nh_bf, nw_bf)
    zqout, parts = _sc_call(wb, idx, h_batch)
    loss = (1.0 + _BETA) * (1.0 / (_B * _D)) * jnp.sum(parts)
    return (zqout, loss, perp[0, 0], onehot, idx)


# speedup vs baseline: 1.1467x; 1.1467x over previous
import functools

import jax
import jax.numpy as jnp
from jax import lax
from jax.experimental import pallas as pl
from jax.experimental.pallas import tpu as pltpu
from jax.experimental.pallas import tpu_sc as plsc

_B = 16384
_K = 8192
_D = 64
_TB = 256
_BETA = 0.25

_NC = 2
_NS = 16
_NW = _NC * _NS
_RW = _B // _NW
_CH = 128


def _normalize_rows(x, eps=1e-12):
    norm = jnp.linalg.norm(x, axis=1, keepdims=True)
    return x / jnp.maximum(norm, eps)


def _tc_body(nh_ref, nw_ref,
             onehot_ref, idx_ref, perp_ref,
             counts_ref):
    b = pl.program_id(0)
    nb = pl.num_programs(0)

    d = jax.lax.dot_general(
        nh_ref[...], nw_ref[...], (((1,), (1,)), ((), ())),
        preferred_element_type=jnp.float32)
    m = jnp.max(d, axis=1, keepdims=True)
    col = jax.lax.broadcasted_iota(jnp.int32, (_TB, _K), 1)
    sel = jnp.where(d >= m, col, -1)
    idx = jnp.max(sel, axis=1)
    idx_ref[...] = idx
    oh = (col == idx[:, None]).astype(jnp.float32)
    onehot_ref[...] = oh

    @pl.when(b == 0)
    def _init():
        counts_ref[...] = jnp.zeros_like(counts_ref)
        perp_ref[...] = jnp.zeros_like(perp_ref)

    counts_ref[...] += jnp.sum(oh, axis=0, keepdims=True)

    @pl.when(b == nb - 1)
    def _fin():
        p = counts_ref[...] * (1.0 / _B)
        ent = jnp.sum(p * jnp.log(p + 1e-10))
        perp_ref[...] = jnp.exp(-ent).reshape(1, 1)


def _tc_call(nh_bf, nw_bf):
    return pl.pallas_call(
        _tc_body,
        grid=(_B // _TB,),
        in_specs=[
            pl.BlockSpec((_TB, _D), lambda b: (b, 0)),
            pl.BlockSpec((_K, _D), lambda b: (0, 0)),
        ],
        out_specs=[
            pl.BlockSpec((_TB, _K), lambda b: (b, 0)),
            pl.BlockSpec((_TB,), lambda b: (b,)),
            pl.BlockSpec((1, 1), lambda b: (0, 0)),
        ],
        out_shape=[
            jax.ShapeDtypeStruct((_B, _K), jnp.float32),
            jax.ShapeDtypeStruct((_B,), jnp.int32),
            jax.ShapeDtypeStruct((1, 1), jnp.float32),
        ],
        scratch_shapes=[
            pltpu.VMEM((1, _K), jnp.float32),
        ],
        compiler_params=pltpu.CompilerParams(
            dimension_semantics=("arbitrary",),
        ),
    )(nh_bf, nw_bf)


def _sc_body(wb_hbm, idx_hbm, h_hbm,
             zqout_hbm, part_hbm,
             idx_v, rows_v, h_v, out_v, part_v, sem):
    wid = lax.axis_index("s") * _NC + lax.axis_index("c")
    base = wid * _RW
    pltpu.sync_copy(idx_hbm.at[pl.ds(base, _RW)], idx_v)

    acc0 = jnp.zeros((16,), jnp.float32)

    def chunk(c, acc):
        cb = base + c * _CH
        pltpu.async_copy(
            wb_hbm.at[idx_v.at[pl.ds(c * _CH, _CH)]], rows_v, sem).wait()
        pltpu.sync_copy(h_hbm.at[pl.ds(cb, _CH)], h_v)

        def row(i, a):
            for j in range(_D // 16):
                zq = rows_v[i, pl.ds(j * 16, 16)]
                hh = h_v[i, pl.ds(j * 16, 16)]
                dd = zq - hh
                out_v[i, pl.ds(j * 16, 16)] = hh + dd
                a = a + dd * dd
            return a

        acc = lax.fori_loop(0, _CH, row, acc)
        pltpu.sync_copy(out_v, zqout_hbm.at[pl.ds(cb, _CH)])
        return acc

    acc = lax.fori_loop(0, _RW // _CH, chunk, acc0)
    part_v[...] = acc
    pltpu.sync_copy(part_v, part_hbm.at[wid])


def _sc_call(wb, idx, h_batch):
    sc = functools.partial(
        pl.kernel,
        out_type=[
            jax.ShapeDtypeStruct((_B, _D), jnp.float32),
            jax.ShapeDtypeStruct((_NW, 16), jnp.float32),
        ],
        mesh=plsc.VectorSubcoreMesh(core_axis_name="c",
                                    subcore_axis_name="s"),
        scratch_types=[
            pltpu.VMEM((_RW,), jnp.int32),
            pltpu.VMEM((_CH, 128), jnp.float32),
            pltpu.VMEM((_CH, _D), jnp.float32),
            pltpu.VMEM((_CH, _D), jnp.float32),
            pltpu.VMEM((16,), jnp.float32),
            pltpu.SemaphoreType.DMA,
        ],
    )(_sc_body)
    return sc(wb, idx, h_batch)


def kernel(h_batch, W):
    nh = _normalize_rows(jax.lax.stop_gradient(h_batch))
    nw = _normalize_rows(W)
    nh_bf = nh.astype(jnp.bfloat16)
    nw_bf = nw.astype(jnp.bfloat16)
    wb = W.astype(jnp.bfloat16).astype(jnp.float32)
    wb = jnp.concatenate([wb, jnp.zeros((_K, 128 - _D), jnp.float32)], axis=1)

    onehot, idx, perp = _tc_call(nh_bf, nw_bf)
    zqout, parts = _sc_call(wb, idx, h_batch)
    loss = (1.0 + _BETA) * (1.0 / (_B * _D)) * jnp.sum(parts)
    return (zqout, loss, perp[0, 0], onehot, idx)
